# baseline (device time: 155459 ns/iter reference)
import jax
import jax.numpy as jnp
from jax import lax
from jax.experimental import pallas as pl
from jax.experimental.pallas import tpu as pltpu

N_DEV = 4


def kernel(x, w_mat):
    m, k_per = x.shape
    _, n = w_mat.shape
    m_per = m // N_DEV

    def body(x_ref, w_ref, out_ref, comm_ref, send_sems, recv_sems):
        my = lax.axis_index("i")
        left = lax.rem(my + N_DEV - 1, N_DEV)
        right = lax.rem(my + 1, N_DEV)

        barrier_sem = pltpu.get_barrier_semaphore()
        for nbr in (left, right):
            pl.semaphore_signal(
                barrier_sem, inc=1,
                device_id=(nbr,), device_id_type=pl.DeviceIdType.MESH,
            )
        pl.semaphore_wait(barrier_sem, 2)

        def partial(j):
            xs = x_ref[pl.ds(j * m_per, m_per), :]
            return jnp.dot(xs, w_ref[:, :], preferred_element_type=jnp.float32)

        j0 = lax.rem(my + N_DEV - 1, N_DEV)
        comm_ref[0, :, :] = partial(j0)

        for t in range(1, N_DEV):
            send_slot = (t - 1) % 2
            recv_slot = t % 2
            rdma = pltpu.make_async_remote_copy(
                src_ref=comm_ref.at[send_slot],
                dst_ref=comm_ref.at[recv_slot],
                send_sem=send_sems.at[send_slot],
                recv_sem=recv_sems.at[recv_slot],
                device_id=(right,),
                device_id_type=pl.DeviceIdType.MESH,
            )
            rdma.start()
            rdma.wait()

            j = lax.rem(my + N_DEV - 1 - t + N_DEV, N_DEV)
            p = partial(j) + comm_ref[recv_slot, :, :]
            if t < N_DEV - 1:
                comm_ref[recv_slot, :, :] = p
            else:
                out_ref[:, :] = p * jax.nn.sigmoid(p)

    return pl.pallas_call(
        body,
        out_shape=jax.ShapeDtypeStruct((m_per, n), jnp.float32),
        in_specs=[
            pl.BlockSpec(memory_space=pltpu.VMEM),
            pl.BlockSpec(memory_space=pltpu.VMEM),
        ],
        out_specs=pl.BlockSpec(memory_space=pltpu.VMEM),
        scratch_shapes=[
            pltpu.VMEM((2, m_per, n), jnp.float32),
            pltpu.SemaphoreType.DMA((2,)),
            pltpu.SemaphoreType.DMA((2,)),
        ],
        compiler_params=pltpu.CompilerParams(collective_id=0),
    )(x, w_mat)


# device time: 88263 ns/iter; 1.7613x vs baseline; 1.7613x over previous
import jax
import jax.numpy as jnp
from jax import lax
from jax.experimental import pallas as pl
from jax.experimental.pallas import tpu as pltpu

N_DEV = 4


def kernel(x, w_mat):
    m, k_per = x.shape
    _, n = w_mat.shape
    m_per = m // N_DEV
    n_half = n // 2

    def body(x_ref, w_ref, out_ref,
             comm_r, comm_l, send_r, recv_r, send_l, recv_l):
        my = lax.axis_index("i")
        left = lax.rem(my + N_DEV - 1, N_DEV)
        right = lax.rem(my + 1, N_DEV)

        barrier_sem = pltpu.get_barrier_semaphore()
        for nbr in (left, right):
            pl.semaphore_signal(
                barrier_sem, inc=1,
                device_id=(nbr,), device_id_type=pl.DeviceIdType.MESH,
            )
        pl.semaphore_wait(barrier_sem, 2)

        def partial_r(j):
            xs = x_ref[pl.ds(j * m_per, m_per), :]
            return jnp.dot(xs, w_ref[:, :n_half],
                           preferred_element_type=jnp.float32)

        def partial_l(j):
            xs = x_ref[pl.ds(j * m_per, m_per), :]
            return jnp.dot(xs, w_ref[:, n_half:],
                           preferred_element_type=jnp.float32)

        comm_r[0, :, :] = partial_r(lax.rem(my + N_DEV - 1, N_DEV))
        comm_l[0, :, :] = partial_l(lax.rem(my + 1, N_DEV))

        for t in range(1, N_DEV):
            send_slot = (t - 1) % 2
            recv_slot = t % 2
            rdma_r = pltpu.make_async_remote_copy(
                src_ref=comm_r.at[send_slot],
                dst_ref=comm_r.at[recv_slot],
                send_sem=send_r.at[send_slot],
                recv_sem=recv_r.at[recv_slot],
                device_id=(right,),
                device_id_type=pl.DeviceIdType.MESH,
            )
            rdma_l = pltpu.make_async_remote_copy(
                src_ref=comm_l.at[send_slot],
                dst_ref=comm_l.at[recv_slot],
                send_sem=send_l.at[send_slot],
                recv_sem=recv_l.at[recv_slot],
                device_id=(left,),
                device_id_type=pl.DeviceIdType.MESH,
            )
            rdma_r.start()
            rdma_l.start()

            p_r = partial_r(lax.rem(my + 2 * N_DEV - 1 - t, N_DEV))
            p_l = partial_l(lax.rem(my + 1 + t, N_DEV))

            rdma_r.wait()
            rdma_l.wait()

            if t < N_DEV - 1:
                comm_r[recv_slot, :, :] = p_r + comm_r[recv_slot, :, :]
                comm_l[recv_slot, :, :] = p_l + comm_l[recv_slot, :, :]
            else:
                y_r = p_r + comm_r[recv_slot, :, :]
                y_l = p_l + comm_l[recv_slot, :, :]
                out_ref[:, :n_half] = y_r * jax.nn.sigmoid(y_r)
                out_ref[:, n_half:] = y_l * jax.nn.sigmoid(y_l)

    return pl.pallas_call(
        body,
        out_shape=jax.ShapeDtypeStruct((m_per, n), jnp.float32),
        in_specs=[
            pl.BlockSpec(memory_space=pltpu.VMEM),
            pl.BlockSpec(memory_space=pltpu.VMEM),
        ],
        out_specs=pl.BlockSpec(memory_space=pltpu.VMEM),
        scratch_shapes=[
            pltpu.VMEM((2, m_per, n_half), jnp.float32),
            pltpu.VMEM((2, m_per, n_half), jnp.float32),
            pltpu.SemaphoreType.DMA((2,)),
            pltpu.SemaphoreType.DMA((2,)),
            pltpu.SemaphoreType.DMA((2,)),
            pltpu.SemaphoreType.DMA((2,)),
        ],
        compiler_params=pltpu.CompilerParams(collective_id=0),
    )(x, w_mat)


# device time: 80067 ns/iter; 1.9416x vs baseline; 1.1024x over previous
import jax
import jax.numpy as jnp
from jax import lax
from jax.experimental import pallas as pl
from jax.experimental.pallas import tpu as pltpu

N_DEV = 4


def kernel(x, w_mat):
    m, k_per = x.shape
    _, n = w_mat.shape
    m_per = m // N_DEV
    n_half = n // 2
    n_q = n_half // 2

    def body(x_ref, w_ref, out_ref,
             comm_r, comm_l, send_r, recv_r, send_l, recv_l):
        my = lax.axis_index("i")
        left = lax.rem(my + N_DEV - 1, N_DEV)
        right = lax.rem(my + 1, N_DEV)

        barrier_sem = pltpu.get_barrier_semaphore()
        for nbr in (left, right):
            pl.semaphore_signal(
                barrier_sem, inc=1,
                device_id=(nbr,), device_id_type=pl.DeviceIdType.MESH,
            )
        pl.semaphore_wait(barrier_sem, 2)

        def xs(j):
            return x_ref[pl.ds(j * m_per, m_per), :]

        def col_base(dir_r, piece):
            return (0 if dir_r else n_half) + piece * n_q

        def mk(dir_r, piece, src_slot, dst_slot):
            comm = comm_r if dir_r else comm_l
            ssem = send_r if dir_r else send_l
            rsem = recv_r if dir_r else recv_l
            tgt = right if dir_r else left
            return pltpu.make_async_remote_copy(
                src_ref=comm.at[src_slot, piece],
                dst_ref=comm.at[dst_slot, piece],
                send_sem=ssem.at[src_slot, piece],
                recv_sem=rsem.at[dst_slot, piece],
                device_id=(tgt,),
                device_id_type=pl.DeviceIdType.MESH,
            )

        PIECES = ((True, 0), (False, 0), (True, 1), (False, 1))
        sends = {}

        jr0 = lax.rem(my + N_DEV - 1, N_DEV)
        jl0 = lax.rem(my + 1, N_DEV)
        for dir_r, piece in PIECES:
            j = jr0 if dir_r else jl0
            comm = comm_r if dir_r else comm_l
            b = col_base(dir_r, piece)
            comm[0, piece] = jnp.dot(
                xs(j), w_ref[:, b:b + n_q],
                preferred_element_type=jnp.float32)
            d = mk(dir_r, piece, 0, 1)
            d.start()
            sends[(1, dir_r, piece)] = d

        for t in range(1, N_DEV):
            ss, rs = (t - 1) % 2, t % 2
            jr = lax.rem(my + 2 * N_DEV - 1 - t, N_DEV)
            jl = lax.rem(my + 1 + t, N_DEV)
            p_r = jnp.dot(xs(jr), w_ref[:, :n_half],
                          preferred_element_type=jnp.float32)
            p_l = jnp.dot(xs(jl), w_ref[:, n_half:],
                          preferred_element_type=jnp.float32)

            for dir_r, piece in PIECES:
                comm = comm_r if dir_r else comm_l
                p = p_r if dir_r else p_l
                mk(dir_r, piece, ss, rs).wait_recv()
                acc = p[:, piece * n_q:(piece + 1) * n_q] + comm[rs, piece]
                if t < N_DEV - 1:
                    prev = sends.pop((t - 1, dir_r, piece), None)
                    if prev is not None:
                        prev.wait_send()
                    comm[rs, piece] = acc
                    d = mk(dir_r, piece, rs, ss)
                    d.start()
                    sends[(t + 1, dir_r, piece)] = d
                else:
                    b = col_base(dir_r, piece)
                    out_ref[:, b:b + n_q] = acc * jax.nn.sigmoid(acc)

        for d in sends.values():
            d.wait_send()

    return pl.pallas_call(
        body,
        out_shape=jax.ShapeDtypeStruct((m_per, n), jnp.float32),
        in_specs=[
            pl.BlockSpec(memory_space=pltpu.VMEM),
            pl.BlockSpec(memory_space=pltpu.VMEM),
        ],
        out_specs=pl.BlockSpec(memory_space=pltpu.VMEM),
        scratch_shapes=[
            pltpu.VMEM((2, 2, m_per, n_q), jnp.float32),
            pltpu.VMEM((2, 2, m_per, n_q), jnp.float32),
            pltpu.SemaphoreType.DMA((2, 2)),
            pltpu.SemaphoreType.DMA((2, 2)),
            pltpu.SemaphoreType.DMA((2, 2)),
            pltpu.SemaphoreType.DMA((2, 2)),
        ],
        compiler_params=pltpu.CompilerParams(collective_id=0),
    )(x, w_mat)


# device time: 79904 ns/iter; 1.9456x vs baseline; 1.0020x over previous
import jax
import jax.numpy as jnp
from jax import lax
from jax.experimental import pallas as pl
from jax.experimental.pallas import tpu as pltpu

N_DEV = 4
N_PIECES = 4


def kernel(x, w_mat):
    m, k_per = x.shape
    _, n = w_mat.shape
    m_per = m // N_DEV
    n_half = n // 2
    n_q = n_half // N_PIECES

    def body(x_ref, w_ref, out_ref,
             comm_r, comm_l, send_r, recv_r, send_l, recv_l):
        my = lax.axis_index("i")
        left = lax.rem(my + N_DEV - 1, N_DEV)
        right = lax.rem(my + 1, N_DEV)

        barrier_sem = pltpu.get_barrier_semaphore()
        for nbr in (left, right):
            pl.semaphore_signal(
                barrier_sem, inc=1,
                device_id=(nbr,), device_id_type=pl.DeviceIdType.MESH,
            )
        pl.semaphore_wait(barrier_sem, 2)

        def xs(j):
            return x_ref[pl.ds(j * m_per, m_per), :]

        def col_base(dir_r, piece):
            return (0 if dir_r else n_half) + piece * n_q

        def mk(dir_r, piece, src_slot, dst_slot):
            comm = comm_r if dir_r else comm_l
            ssem = send_r if dir_r else send_l
            rsem = recv_r if dir_r else recv_l
            tgt = right if dir_r else left
            return pltpu.make_async_remote_copy(
                src_ref=comm.at[src_slot, piece],
                dst_ref=comm.at[dst_slot, piece],
                send_sem=ssem.at[src_slot, piece],
                recv_sem=rsem.at[dst_slot, piece],
                device_id=(tgt,),
                device_id_type=pl.DeviceIdType.MESH,
            )

        PIECES = tuple(
            (dir_r, p) for p in range(N_PIECES) for dir_r in (True, False)
        )
        sends = {}

        jr0 = lax.rem(my + N_DEV - 1, N_DEV)
        jl0 = lax.rem(my + 1, N_DEV)
        for dir_r, piece in PIECES:
            j = jr0 if dir_r else jl0
            comm = comm_r if dir_r else comm_l
            b = col_base(dir_r, piece)
            comm[0, piece] = jnp.dot(
                xs(j), w_ref[:, b:b + n_q],
                preferred_element_type=jnp.float32)
            d = mk(dir_r, piece, 0, 1)
            d.start()
            sends[(1, dir_r, piece)] = d

        for t in range(1, N_DEV):
            ss, rs = (t - 1) % 2, t % 2
            jr = lax.rem(my + 2 * N_DEV - 1 - t, N_DEV)
            jl = lax.rem(my + 1 + t, N_DEV)
            p_r = jnp.dot(xs(jr), w_ref[:, :n_half],
                          preferred_element_type=jnp.float32)
            p_l = jnp.dot(xs(jl), w_ref[:, n_half:],
                          preferred_element_type=jnp.float32)

            for dir_r, piece in PIECES:
                comm = comm_r if dir_r else comm_l
                p = p_r if dir_r else p_l
                mk(dir_r, piece, ss, rs).wait_recv()
                acc = p[:, piece * n_q:(piece + 1) * n_q] + comm[rs, piece]
                if t < N_DEV - 1:
                    prev = sends.pop((t - 1, dir_r, piece), None)
                    if prev is not None:
                        prev.wait_send()
                    comm[rs, piece] = acc
                    d = mk(dir_r, piece, rs, ss)
                    d.start()
                    sends[(t + 1, dir_r, piece)] = d
                else:
                    b = col_base(dir_r, piece)
                    out_ref[:, b:b + n_q] = acc * jax.nn.sigmoid(acc)

        for d in sends.values():
            d.wait_send()

    return pl.pallas_call(
        body,
        out_shape=jax.ShapeDtypeStruct((m_per, n), jnp.float32),
        in_specs=[
            pl.BlockSpec(memory_space=pltpu.VMEM),
            pl.BlockSpec(memory_space=pltpu.VMEM),
        ],
        out_specs=pl.BlockSpec(memory_space=pltpu.VMEM),
        scratch_shapes=[
            pltpu.VMEM((2, N_PIECES, m_per, n_q), jnp.float32),
            pltpu.VMEM((2, N_PIECES, m_per, n_q), jnp.float32),
            pltpu.SemaphoreType.DMA((2, N_PIECES)),
            pltpu.SemaphoreType.DMA((2, N_PIECES)),
            pltpu.SemaphoreType.DMA((2, N_PIECES)),
            pltpu.SemaphoreType.DMA((2, N_PIECES)),
        ],
        compiler_params=pltpu.CompilerParams(collective_id=0),
    )(x, w_mat)


# device time: 46175 ns/iter; 3.3667x vs baseline; 1.7305x over previous
import jax
import jax.numpy as jnp
from jax import lax
from jax.experimental import pallas as pl
from jax.experimental.pallas import tpu as pltpu

N_DEV = 4
N_PIECES = 4


def kernel(x, w_mat):
    m, k_per = x.shape
    _, n = w_mat.shape
    m_per = m // N_DEV
    n_half = n // 2
    n_q = n_half // N_PIECES

    def body(x_ref, w_ref, out_ref,
             comm_r, comm_l, send_r, recv_r, send_l, recv_l):
        my = lax.axis_index("i")
        left = lax.rem(my + N_DEV - 1, N_DEV)
        right = lax.rem(my + 1, N_DEV)

        barrier_sem = pltpu.get_barrier_semaphore()
        for nbr in (left, right):
            pl.semaphore_signal(
                barrier_sem, inc=1,
                device_id=(nbr,), device_id_type=pl.DeviceIdType.MESH,
            )
        pl.semaphore_wait(barrier_sem, 2)

        def xs(j):
            return x_ref[pl.ds(j * m_per, m_per), :]

        def col_base(dir_r, piece):
            return (0 if dir_r else n_half) + piece * n_q

        def mk(dir_r, piece, src_slot, dst_slot):
            comm = comm_r if dir_r else comm_l
            ssem = send_r if dir_r else send_l
            rsem = recv_r if dir_r else recv_l
            tgt = right if dir_r else left
            return pltpu.make_async_remote_copy(
                src_ref=comm.at[src_slot, piece],
                dst_ref=comm.at[dst_slot, piece],
                send_sem=ssem.at[src_slot, piece],
                recv_sem=rsem.at[dst_slot, piece],
                device_id=(tgt,),
                device_id_type=pl.DeviceIdType.MESH,
            )

        PIECES = tuple(
            (dir_r, p) for p in range(N_PIECES) for dir_r in (True, False)
        )
        sends = {}

        jr0 = lax.rem(my + N_DEV - 1, N_DEV)
        jl0 = lax.rem(my + 1, N_DEV)
        for dir_r, piece in PIECES:
            j = jr0 if dir_r else jl0
            comm = comm_r if dir_r else comm_l
            b = col_base(dir_r, piece)
            comm[0, piece] = jnp.dot(
                xs(j), w_ref[:, b:b + n_q],
                preferred_element_type=jnp.float32).astype(jnp.bfloat16)
            d = mk(dir_r, piece, 0, 1)
            d.start()
            sends[(1, dir_r, piece)] = d

        for t in range(1, N_DEV):
            ss, rs = (t - 1) % 2, t % 2
            jr = lax.rem(my + 2 * N_DEV - 1 - t, N_DEV)
            jl = lax.rem(my + 1 + t, N_DEV)
            p_r = jnp.dot(xs(jr), w_ref[:, :n_half],
                          preferred_element_type=jnp.float32)
            p_l = jnp.dot(xs(jl), w_ref[:, n_half:],
                          preferred_element_type=jnp.float32)

            for dir_r, piece in PIECES:
                comm = comm_r if dir_r else comm_l
                p = p_r if dir_r else p_l
                mk(dir_r, piece, ss, rs).wait_recv()
                acc = (p[:, piece * n_q:(piece + 1) * n_q]
                       + comm[rs, piece].astype(jnp.float32))
                if t < N_DEV - 1:
                    prev = sends.pop((t - 1, dir_r, piece), None)
                    if prev is not None:
                        prev.wait_send()
                    comm[rs, piece] = acc.astype(jnp.bfloat16)
                    d = mk(dir_r, piece, rs, ss)
                    d.start()
                    sends[(t + 1, dir_r, piece)] = d
                else:
                    b = col_base(dir_r, piece)
                    out_ref[:, b:b + n_q] = acc * jax.nn.sigmoid(acc)

        for d in sends.values():
            d.wait_send()

    return pl.pallas_call(
        body,
        out_shape=jax.ShapeDtypeStruct((m_per, n), jnp.float32),
        in_specs=[
            pl.BlockSpec(memory_space=pltpu.VMEM),
            pl.BlockSpec(memory_space=pltpu.VMEM),
        ],
        out_specs=pl.BlockSpec(memory_space=pltpu.VMEM),
        scratch_shapes=[
            pltpu.VMEM((2, N_PIECES, m_per, n_q), jnp.bfloat16),
            pltpu.VMEM((2, N_PIECES, m_per, n_q), jnp.bfloat16),
            pltpu.SemaphoreType.DMA((2, N_PIECES)),
            pltpu.SemaphoreType.DMA((2, N_PIECES)),
            pltpu.SemaphoreType.DMA((2, N_PIECES)),
            pltpu.SemaphoreType.DMA((2, N_PIECES)),
        ],
        compiler_params=pltpu.CompilerParams(collective_id=0),
    )(x, w_mat)
